# Initial kernel scaffold; baseline (speedup 1.0000x reference)
import jax
import jax.numpy as jnp
from jax.experimental import pallas as pl


def kernel(q, k, v, Wq, Wk, Wv, Wout):
    raise NotImplementedError("write your pallas kernel here")


# speedup vs baseline: 158.9078x; 158.9078x over previous
import functools

import jax
import jax.numpy as jnp
from jax.experimental import pallas as pl
from jax.experimental.pallas import tpu as pltpu

HEAD_NUM = 12
DIM_QK = 64
DIM_V = 64
SPAN = 32
LQ = 2048
LKV = 2048
DIM = 768

BQ = 256
W = BQ + SPAN
NB = LQ // BQ


def _fused_kernel(q_ref, k_ref, v_ref, wq_ref, wk_ref, wv_ref, wo_ref,
                  out_ref, kp_scr, vp_scr):
    i = pl.program_id(0)

    kp_scr[pl.ds(i * BQ, BQ), :] = jnp.dot(
        k_ref[0], wk_ref[...], preferred_element_type=jnp.float32)
    vp_scr[pl.ds(i * BQ, BQ), :] = jnp.dot(
        v_ref[0], wv_ref[...], preferred_element_type=jnp.float32)

    qp = jnp.dot(q_ref[0], wq_ref[...], preferred_element_type=jnp.float32)

    start = pl.multiple_of(jnp.maximum(i * BQ - SPAN, 0), SPAN)
    kwin = kp_scr[pl.ds(start, W), :]
    vwin = vp_scr[pl.ds(start, W), :]

    gq = i * BQ + jax.lax.broadcasted_iota(jnp.int32, (BQ, W), 0)
    gkv = start + jax.lax.broadcasted_iota(jnp.int32, (BQ, W), 1)
    mask = jnp.logical_and(gkv >= gq - (SPAN - 1), gkv <= gq)

    row_ok = (start + jax.lax.broadcasted_iota(jnp.int32, (W, 1), 0)) < (i + 1) * BQ
    vwin = jnp.where(row_ok, vwin, 0.0)

    scale = 1.0 / (DIM_QK ** 0.5)
    outs = []
    for h in range(HEAD_NUM):
        qh = qp[:, h * DIM_QK:(h + 1) * DIM_QK]
        kh = kwin[:, h * DIM_QK:(h + 1) * DIM_QK]
        s = jax.lax.dot_general(
            qh, kh, (((1,), (1,)), ((), ())),
            preferred_element_type=jnp.float32) * scale
        s = jnp.where(mask, s, -jnp.inf)
        m = jnp.max(s, axis=1, keepdims=True)
        p = jnp.exp(s - m)
        p = p / jnp.sum(p, axis=1, keepdims=True)
        vh = vwin[:, h * DIM_V:(h + 1) * DIM_V]
        outs.append(jnp.dot(p, vh, preferred_element_type=jnp.float32))
    o = jnp.concatenate(outs, axis=1)
    out_ref[0] = jnp.dot(o, wo_ref[...], preferred_element_type=jnp.float32)


@jax.jit
def kernel(q, k, v, Wq, Wk, Wv, Wout):
    batch = q.shape[0]
    blk = lambda: pl.BlockSpec((1, BQ, DIM), lambda i: (0, i, 0))
    wspec = lambda: pl.BlockSpec((DIM, HEAD_NUM * DIM_QK), lambda i: (0, 0))
    out = pl.pallas_call(
        _fused_kernel,
        grid=(NB,),
        in_specs=[blk(), blk(), blk(), wspec(), wspec(), wspec(), wspec()],
        out_specs=blk(),
        out_shape=jax.ShapeDtypeStruct((batch, LQ, DIM), jnp.float32),
        scratch_shapes=[
            pltpu.VMEM((LKV, HEAD_NUM * DIM_QK), jnp.float32),
            pltpu.VMEM((LKV, HEAD_NUM * DIM_V), jnp.float32),
        ],
    )(q, k, v, Wq, Wk, Wv, Wout)
    return out
